# baseline (speedup 1.0000x reference)
import jax
import jax.numpy as jnp
from jax.experimental import pallas as pl


def _softmax_body(x_ref, o_ref):
    x = x_ref[...]
    m = jnp.max(x, axis=-1, keepdims=True)
    e = jnp.exp(x - m)
    o_ref[...] = e / jnp.sum(e, axis=-1, keepdims=True)


def _conv2d(x, w, b):
    out = jax.lax.conv_general_dilated(x, w, window_strides=(1, 1), padding='VALID',
                                       dimension_numbers=('NCHW', 'OIHW', 'NCHW'))
    return out + b[None, :, None, None]


def _tag_conv(x, edge_index, norm, W, b):
    src = edge_index[0]
    dst = edge_index[1]
    out = x @ W[0]
    h = x
    for k in range(1, W.shape[0]):
        msg = norm[:, None] * h[src]
        h = jax.ops.segment_sum(msg, dst, num_segments=x.shape[0])
        out = out + h @ W[k]
    return out + b


def kernel(inputs, edge_index, w1, b1, w2, b2, w3, b3, tag1_w, tag1_b, tag2_w, tag2_b, tag3_w, tag3_b):
    h = _conv2d(inputs, w1, b1)
    h = jax.nn.leaky_relu(h, 0.01)
    h = _conv2d(h, w2, b2)
    h = jax.nn.leaky_relu(h, 0.01)
    h = _conv2d(h, w3, b3)
    h = h.reshape(-1, 94)
    h = jnp.tanh(h)
    src = edge_index[0]
    dst = edge_index[1]
    deg = jax.ops.segment_sum(jnp.ones((src.shape[0],), dtype=jnp.float32), dst, num_segments=h.shape[0])
    dinv = jnp.where(deg > 0, jax.lax.rsqrt(jnp.maximum(deg, 1e-12)), 0.0)
    norm = dinv[src] * dinv[dst]
    h = _tag_conv(h, edge_index, norm, tag1_w, tag1_b)
    h = jnp.tanh(h)
    h = _tag_conv(h, edge_index, norm, tag2_w, tag2_b)
    h = jnp.tanh(h)
    h = _tag_conv(h, edge_index, norm, tag3_w, tag3_b)
    n = h.shape[0]
    return pl.pallas_call(
        _softmax_body,
        out_shape=jax.ShapeDtypeStruct((n, 16), jnp.float32),
        grid=(n // 1000,),
        in_specs=[pl.BlockSpec((1000, 16), lambda i: (i, 0))],
        out_specs=pl.BlockSpec((1000, 16), lambda i: (i, 0)),
    )(h)


# speedup vs baseline: 3.3294x; 3.3294x over previous
import functools

import jax
import jax.numpy as jnp
from jax import lax
from jax.experimental import pallas as pl
from jax.experimental.pallas import tpu as pltpu
from jax.experimental.pallas import tpu_sc as plsc

N = 50000
E = 800000
EPAD = 819200
ACC_ROWS = 50048
S = ACC_ROWS
ZPT = ACC_ROWS // 16
P = 8


def _hop_body(C, table, srcp, dstp, zeros, z, sbuf, dbuf, gbuf, rows, acc,
              sem_i, sem_g, sem_s):
    t = lax.axis_index("s")
    core = lax.axis_index("c")
    ept = EPAD // 16
    iters = ept // (P * 128)
    for cc in range((C + 1) // 2):
        c = 2 * cc + core

        @pl.when(c < C)
        def _():
            cn = c * S
            pltpu.sync_copy(zeros, acc.at[pl.ds(t * ZPT, ZPT)])
            plsc.subcore_barrier()

            def body(j, carry):
                ebase = t * ept + j * (P * 128)
                d_i = []
                for p in range(P):
                    d_i.append(pltpu.async_copy(
                        srcp.at[pl.ds(ebase + p * 128, 128)], sbuf.at[p], sem_i))
                    d_i.append(pltpu.async_copy(
                        dstp.at[pl.ds(ebase + p * 128, 128)], dbuf.at[p], sem_i))
                for d in d_i:
                    d.wait()
                for p in range(P):
                    for k in range(8):
                        gbuf[p, pl.ds(k * 16, 16)] = sbuf[p, pl.ds(k * 16, 16)] + cn
                d_g = [pltpu.async_copy(table.at[gbuf.at[p]], rows.at[p], sem_g)
                       for p in range(P)]
                for d in d_g:
                    d.wait()
                d_s = [pltpu.async_copy(rows.at[p], acc.at[dbuf.at[p]], sem_s,
                                        add=True)
                       for p in range(P)]
                for d in d_s:
                    d.wait()
                return carry

            lax.fori_loop(0, iters, body, 0)
            plsc.subcore_barrier()
            pltpu.sync_copy(acc.at[pl.ds(t * ZPT, ZPT)],
                            z.at[pl.ds(cn + t * ZPT, ZPT)])
            plsc.subcore_barrier()


@functools.lru_cache(maxsize=None)
def _make_hop(C, F):
    mesh = plsc.VectorSubcoreMesh(core_axis_name="c", subcore_axis_name="s")
    return pl.kernel(
        functools.partial(_hop_body, C),
        out_type=jax.ShapeDtypeStruct((C * S, F), jnp.float32),
        mesh=mesh,
        compiler_params=pltpu.CompilerParams(use_tc_tiling_on_sc=False),
        scratch_types=[
            pltpu.VMEM((P, 128), jnp.int32),
            pltpu.VMEM((P, 128), jnp.int32),
            pltpu.VMEM((P, 128), jnp.int32),
            pltpu.VMEM((P, 128, F), jnp.float32),
            pltpu.VMEM_SHARED((ACC_ROWS, F), jnp.float32),
            pltpu.SemaphoreType.DMA,
            pltpu.SemaphoreType.DMA,
            pltpu.SemaphoreType.DMA,
        ],
    )


def _deg_body(dstp, zeros, ones, degz, dbuf, ones_v, acc, sem_i, sem_s):
    t = lax.axis_index("s")
    core = lax.axis_index("c")
    ept = EPAD // 32
    iters = ept // (P * 128)
    pltpu.sync_copy(zeros, acc.at[pl.ds(t * ZPT, ZPT)])
    pltpu.sync_copy(ones, ones_v)
    plsc.subcore_barrier()

    def body(j, carry):
        ebase = core * (EPAD // 2) + t * ept + j * (P * 128)
        d_i = [pltpu.async_copy(dstp.at[pl.ds(ebase + p * 128, 128)],
                                dbuf.at[p], sem_i) for p in range(P)]
        for d in d_i:
            d.wait()
        d_s = [pltpu.async_copy(ones_v, acc.at[dbuf.at[p]], sem_s, add=True)
               for p in range(P)]
        for d in d_s:
            d.wait()
        return carry

    lax.fori_loop(0, iters, body, 0)
    plsc.subcore_barrier()
    pltpu.sync_copy(acc.at[pl.ds(t * ZPT, ZPT)],
                    degz.at[pl.ds(core * S + t * ZPT, ZPT)])


@functools.lru_cache(maxsize=None)
def _make_deg():
    mesh = plsc.VectorSubcoreMesh(core_axis_name="c", subcore_axis_name="s")
    return pl.kernel(
        _deg_body,
        out_type=jax.ShapeDtypeStruct((2 * S, 16), jnp.float32),
        mesh=mesh,
        compiler_params=pltpu.CompilerParams(use_tc_tiling_on_sc=False),
        scratch_types=[
            pltpu.VMEM((P, 128), jnp.int32),
            pltpu.VMEM((128, 16), jnp.float32),
            pltpu.VMEM_SHARED((ACC_ROWS, 16), jnp.float32),
            pltpu.SemaphoreType.DMA,
            pltpu.SemaphoreType.DMA,
        ],
    )


_BM = 1000


def _enc_bodyfn(x_ref, m1_ref, b1_ref, m2_ref, b2_ref, m3_ref, b3_ref, o_ref):
    h = jnp.dot(x_ref[...], m1_ref[...], preferred_element_type=jnp.float32)
    h = h + b1_ref[...]
    h = jnp.where(h > 0, h, 0.01 * h)
    h = jnp.dot(h, m2_ref[...], preferred_element_type=jnp.float32)
    h = h + b2_ref[...]
    h = jnp.where(h > 0, h, 0.01 * h)
    h = jnp.dot(h, m3_ref[...], preferred_element_type=jnp.float32)
    o_ref[...] = jnp.tanh(h + b3_ref[...])


def _encoder(xf, m1, b1v, m2, b2v, m3, b3v):
    bm = 400
    nb = N // bm
    full = lambda shape: pl.BlockSpec(shape, lambda i: tuple(0 for _ in shape))
    return pl.pallas_call(
        _enc_bodyfn,
        out_shape=jax.ShapeDtypeStruct((N, 96), jnp.float32),
        grid=(nb,),
        in_specs=[
            pl.BlockSpec((bm, 408), lambda i: (i, 0)),
            full(m1.shape), full(b1v.shape), full(m2.shape), full(b2v.shape),
            full(m3.shape), full(b3v.shape),
        ],
        out_specs=pl.BlockSpec((bm, 96), lambda i: (i, 0)),
        compiler_params=pltpu.CompilerParams(
            vmem_limit_bytes=100 * 1024 * 1024),
    )(xf, m1, b1v, m2, b2v, m3, b3v)


def _dinv_bodyfn(a_ref, b_ref, di_ref, di2_ref):
    d = a_ref[:, 0:1] + b_ref[:, 0:1]
    inv = jnp.where(d > 0, lax.rsqrt(jnp.maximum(d, 1e-12)), 0.0)
    di_ref[...] = inv
    di2_ref[...] = inv * inv


def _dinv(dega, degb):
    return pl.pallas_call(
        _dinv_bodyfn,
        out_shape=[jax.ShapeDtypeStruct((N, 1), jnp.float32),
                   jax.ShapeDtypeStruct((N, 1), jnp.float32)],
        grid=(N // _BM,),
        in_specs=[pl.BlockSpec((_BM, 16), lambda i: (i, 0)),
                  pl.BlockSpec((_BM, 16), lambda i: (i, 0))],
        out_specs=[pl.BlockSpec((_BM, 1), lambda i: (i, 0)),
                   pl.BlockSpec((_BM, 1), lambda i: (i, 0))],
    )(dega, degb)


def _prep_bodyfn(C, F, x_ref, d_ref, t_ref):
    xs = x_ref[...] * d_ref[...]
    for c in range(C):
        t_ref[c] = xs[:, c * F:(c + 1) * F]


def _prep0(xp, dinv, C, F):
    return pl.pallas_call(
        functools.partial(_prep_bodyfn, C, F),
        out_shape=jax.ShapeDtypeStruct((C, S, F), jnp.float32),
        grid=(N // _BM,),
        in_specs=[pl.BlockSpec((_BM, C * F), lambda i: (i, 0)),
                  pl.BlockSpec((_BM, 1), lambda i: (i, 0))],
        out_specs=pl.BlockSpec((C, _BM, F), lambda i: (0, i, 0)),
    )(xp, dinv)


def _scale2_bodyfn(z_ref, d_ref, d2_ref, h_ref, t_ref):
    h_ref[0] = z_ref[0] * d_ref[...]
    t_ref[0] = z_ref[0] * d2_ref[...]


def _scale1_bodyfn(z_ref, d_ref, h_ref):
    h_ref[0] = z_ref[0] * d_ref[...]


def _scale(z, dinv, dinv2, C, F):
    spec3 = pl.BlockSpec((1, _BM, F), lambda c, i: (c, i, 0))
    spec1 = pl.BlockSpec((_BM, 1), lambda c, i: (i, 0))
    if dinv2 is None:
        return pl.pallas_call(
            _scale1_bodyfn,
            out_shape=jax.ShapeDtypeStruct((C, S, F), jnp.float32),
            grid=(C, N // _BM),
            in_specs=[spec3, spec1],
            out_specs=spec3,
        )(z, dinv)
    return pl.pallas_call(
        _scale2_bodyfn,
        out_shape=[jax.ShapeDtypeStruct((C, S, F), jnp.float32),
                   jax.ShapeDtypeStruct((C, S, F), jnp.float32)],
        grid=(C, N // _BM),
        in_specs=[spec3, spec1, spec1],
        out_specs=[spec3, spec3],
    )(z, dinv, dinv2)


def _mm_bodyfn(x_ref, w_ref, o_ref):
    o_ref[...] = jnp.dot(x_ref[...], w_ref[...],
                         preferred_element_type=jnp.float32)


def _dense_mm(x, w):
    k, dout = w.shape
    return pl.pallas_call(
        _mm_bodyfn,
        out_shape=jax.ShapeDtypeStruct((N, dout), jnp.float32),
        grid=(N // _BM,),
        in_specs=[pl.BlockSpec((_BM, k), lambda i: (i, 0)),
                  pl.BlockSpec((k, dout), lambda i: (0, 0))],
        out_specs=pl.BlockSpec((_BM, dout), lambda i: (i, 0)),
    )(x, w)


def _cmm_bodyfn(C, h_ref, w_ref, o_ref):
    acc = jnp.dot(h_ref[0], w_ref[0], preferred_element_type=jnp.float32)
    for c in range(1, C):
        acc = acc + jnp.dot(h_ref[c], w_ref[c],
                            preferred_element_type=jnp.float32)
    o_ref[...] = acc


def _chunk_mm(h, wc):
    C, F, dout = wc.shape
    return pl.pallas_call(
        functools.partial(_cmm_bodyfn, C),
        out_shape=jax.ShapeDtypeStruct((N, dout), jnp.float32),
        grid=(N // _BM,),
        in_specs=[pl.BlockSpec((C, _BM, F), lambda i: (0, i, 0)),
                  pl.BlockSpec((C, F, dout), lambda i: (0, 0, 0))],
        out_specs=pl.BlockSpec((_BM, dout), lambda i: (i, 0)),
    )(h, wc)


def _comb_tanh_bodyfn(a_ref, b_ref, c_ref, bias_ref, o_ref):
    o_ref[...] = jnp.tanh(a_ref[...] + b_ref[...] + c_ref[...] + bias_ref[...])


def _comb_smax_bodyfn(a_ref, b_ref, c_ref, bias_ref, o_ref):
    x = a_ref[...] + b_ref[...] + c_ref[...] + bias_ref[...]
    m = jnp.max(x, axis=-1, keepdims=True)
    e = jnp.exp(x - m)
    o_ref[...] = e / jnp.sum(e, axis=-1, keepdims=True)


def _combine(p0, p1, p2, bias, softmax):
    dout = p0.shape[1]
    body = _comb_smax_bodyfn if softmax else _comb_tanh_bodyfn
    return pl.pallas_call(
        body,
        out_shape=jax.ShapeDtypeStruct((N, dout), jnp.float32),
        grid=(N // _BM,),
        in_specs=[pl.BlockSpec((_BM, dout), lambda i: (i, 0))] * 3
                 + [pl.BlockSpec((1, dout), lambda i: (0, 0))],
        out_specs=pl.BlockSpec((_BM, dout), lambda i: (i, 0)),
    )(p0, p1, p2, bias)


def _band(hi, i, a):
    return (jnp.arange(hi)[:, None, None] - jnp.arange(i)[None, :, None]
            == jnp.arange(a)[None, None, :]).astype(jnp.float32)


def _conv_mats(w1, b1, w2, b2, w3, b3):
    m1 = jnp.einsum('hia,wjb,oab->hwoij', _band(8, 6, 3), _band(51, 49, 3),
                    w1[:, 0]).reshape(408, 2940)
    m2 = jnp.einsum('hia,wjb,ocab->chwoij', _band(6, 4, 3), _band(49, 48, 2),
                    w2).reshape(2940, 3840)
    m3 = jnp.einsum('hia,wjb,ocab->chwoij', _band(4, 2, 3), _band(48, 47, 2),
                    w3).reshape(3840, 94)
    m3 = jnp.pad(m3, ((0, 0), (0, 2)))
    b1v = jnp.repeat(b1, 294)[None]
    b2v = jnp.repeat(b2, 192)[None]
    b3v = jnp.pad(jnp.repeat(b3, 94), (0, 2))[None]
    return m1, b1v, m2, b2v, m3, b3v


def _tag_layer(x, dinv, dinv2, w, b, srcp, dstp, zeros, softmax):
    din, dout = w.shape[1], w.shape[2]
    F = 16
    C = (din + F - 1) // F
    wp = jnp.pad(w, ((0, 0), (0, C * F - din), (0, 0)))
    hop = _make_hop(C, F)
    p0 = _dense_mm(x, wp[0])
    table0 = _prep0(x, dinv, C, F)
    z1 = hop(table0.reshape(C * S, F), srcp, dstp, zeros)
    h1, table1 = _scale(z1.reshape(C, S, F), dinv, dinv2, C, F)
    p1 = _chunk_mm(h1, wp[1].reshape(C, F, dout))
    z2 = hop(table1.reshape(C * S, F), srcp, dstp, zeros)
    h2 = _scale(z2.reshape(C, S, F), dinv, None, C, F)
    p2 = _chunk_mm(h2, wp[2].reshape(C, F, dout))
    return _combine(p0, p1, p2, b[None], softmax)


def kernel(inputs, edge_index, w1, b1, w2, b2, w3, b3,
           tag1_w, tag1_b, tag2_w, tag2_b, tag3_w, tag3_b):
    src = edge_index[0]
    dst = edge_index[1]
    srcp = jnp.concatenate([src, jnp.zeros((EPAD - E,), jnp.int32)])
    dstp = jnp.concatenate([dst, jnp.full((EPAD - E,), N, jnp.int32)])
    zeros16 = jnp.zeros((ZPT, 16), jnp.float32)
    ones16 = jnp.ones((128, 16), jnp.float32)

    degz = _make_deg()(dstp, zeros16, ones16)
    dinv, dinv2 = _dinv(degz[:N], degz[S:S + N])

    m1, b1v, m2, b2v, m3, b3v = _conv_mats(w1, b1, w2, b2, w3, b3)
    x = _encoder(inputs.reshape(N, 408), m1, b1v, m2, b2v, m3, b3v)

    t1w = jnp.pad(tag1_w, ((0, 0), (0, 2), (0, 0)))
    x = _tag_layer(x, dinv, dinv2, t1w, tag1_b, srcp, dstp, zeros16, False)
    x = _tag_layer(x, dinv, dinv2, tag2_w, tag2_b, srcp, dstp, zeros16, False)
    return _tag_layer(x, dinv, dinv2, tag3_w, tag3_b, srcp, dstp, zeros16, True)


# speedup vs baseline: 3.6845x; 1.1066x over previous
import functools

import jax
import jax.numpy as jnp
from jax import lax
from jax.experimental import pallas as pl
from jax.experimental.pallas import tpu as pltpu
from jax.experimental.pallas import tpu_sc as plsc

N = 50000
E = 800000
EPAD = 819200
ACC_ROWS = 50048
S = ACC_ROWS
ZPT = ACC_ROWS // 16
P = 8


def _hop_body(C, table, eip, zeros, z, sbuf, dbuf, gbuf, rows, acc,
              sem_i, sem_g, sem_s):
    t = lax.axis_index("s")
    core = lax.axis_index("c")
    ept = EPAD // 16
    iters = ept // (P * 128)
    for cc in range((C + 1) // 2):
        c = 2 * cc + core

        @pl.when(c < C)
        def _():
            cn = c * S
            pltpu.sync_copy(zeros, acc.at[pl.ds(t * ZPT, ZPT)])
            plsc.subcore_barrier()

            def body(j, carry):
                ebase = t * ept + j * (P * 128)
                d_i = []
                for p in range(P):
                    d_i.append(pltpu.async_copy(
                        eip.at[0, pl.ds(ebase + p * 128, 128)], sbuf.at[p],
                        sem_i))
                    d_i.append(pltpu.async_copy(
                        eip.at[1, pl.ds(ebase + p * 128, 128)], dbuf.at[p],
                        sem_i))
                d_g = []
                for p in range(P):
                    d_i[2 * p].wait()
                    d_i[2 * p + 1].wait()
                    for k in range(8):
                        gbuf[p, pl.ds(k * 16, 16)] = (
                            sbuf[p, pl.ds(k * 16, 16)] + cn)
                    d_g.append(pltpu.async_copy(
                        table.at[gbuf.at[p]], rows.at[p], sem_g))
                d_s = []
                for p in range(P):
                    d_g[p].wait()
                    d_s.append(pltpu.async_copy(
                        rows.at[p], acc.at[dbuf.at[p]], sem_s, add=True))
                for d in d_s:
                    d.wait()
                return carry

            lax.fori_loop(0, iters, body, 0)
            plsc.subcore_barrier()
            pltpu.sync_copy(acc.at[pl.ds(t * ZPT, ZPT)],
                            z.at[pl.ds(cn + t * ZPT, ZPT)])
            plsc.subcore_barrier()


@functools.lru_cache(maxsize=None)
def _make_hop(C, F):
    mesh = plsc.VectorSubcoreMesh(core_axis_name="c", subcore_axis_name="s")
    return pl.kernel(
        functools.partial(_hop_body, C),
        out_type=jax.ShapeDtypeStruct((C * S, F), jnp.float32),
        mesh=mesh,
        compiler_params=pltpu.CompilerParams(use_tc_tiling_on_sc=False),
        scratch_types=[
            pltpu.VMEM((P, 128), jnp.int32),
            pltpu.VMEM((P, 128), jnp.int32),
            pltpu.VMEM((P, 128), jnp.int32),
            pltpu.VMEM((P, 128, F), jnp.float32),
            pltpu.VMEM_SHARED((ACC_ROWS, F), jnp.float32),
            pltpu.SemaphoreType.DMA,
            pltpu.SemaphoreType.DMA,
            pltpu.SemaphoreType.DMA,
        ],
    )


def _deg_body(eip, zeros, ones, degz, dbuf, ones_v, acc, sem_i, sem_s):
    t = lax.axis_index("s")
    core = lax.axis_index("c")
    ept = EPAD // 32
    iters = ept // (P * 128)
    pltpu.sync_copy(zeros, acc.at[pl.ds(t * ZPT, ZPT)])
    pltpu.sync_copy(ones, ones_v)
    plsc.subcore_barrier()

    def body(j, carry):
        ebase = core * (EPAD // 2) + t * ept + j * (P * 128)
        d_i = [pltpu.async_copy(eip.at[1, pl.ds(ebase + p * 128, 128)],
                                dbuf.at[p], sem_i) for p in range(P)]
        d_s = []
        for p in range(P):
            d_i[p].wait()
            d_s.append(pltpu.async_copy(ones_v, acc.at[dbuf.at[p]], sem_s,
                                        add=True))
        for d in d_s:
            d.wait()
        return carry

    lax.fori_loop(0, iters, body, 0)
    plsc.subcore_barrier()
    pltpu.sync_copy(acc.at[pl.ds(t * ZPT, ZPT)],
                    degz.at[pl.ds(core * S + t * ZPT, ZPT)])


@functools.lru_cache(maxsize=None)
def _make_deg():
    mesh = plsc.VectorSubcoreMesh(core_axis_name="c", subcore_axis_name="s")
    return pl.kernel(
        _deg_body,
        out_type=jax.ShapeDtypeStruct((2 * S, 16), jnp.float32),
        mesh=mesh,
        compiler_params=pltpu.CompilerParams(use_tc_tiling_on_sc=False),
        scratch_types=[
            pltpu.VMEM((P, 128), jnp.int32),
            pltpu.VMEM((128, 16), jnp.float32),
            pltpu.VMEM_SHARED((ACC_ROWS, 16), jnp.float32),
            pltpu.SemaphoreType.DMA,
            pltpu.SemaphoreType.DMA,
        ],
    )


_BM = 1000


def _enc_bodyfn(x_ref, m1_ref, b1_ref, m2_ref, b2_ref, m3_ref, b3_ref, o_ref):
    h = jnp.dot(x_ref[...], m1_ref[...], preferred_element_type=jnp.float32)
    h = h + b1_ref[...]
    h = jnp.where(h > 0, h, 0.01 * h)
    h = jnp.dot(h, m2_ref[...], preferred_element_type=jnp.float32)
    h = h + b2_ref[...]
    h = jnp.where(h > 0, h, 0.01 * h)
    h = jnp.dot(h, m3_ref[...], preferred_element_type=jnp.float32)
    o_ref[...] = jnp.tanh(h + b3_ref[...])


def _encoder(xf, m1, b1v, m2, b2v, m3, b3v):
    bm = 400
    nb = N // bm
    full = lambda shape: pl.BlockSpec(shape, lambda i: tuple(0 for _ in shape))
    return pl.pallas_call(
        _enc_bodyfn,
        out_shape=jax.ShapeDtypeStruct((N, 96), jnp.float32),
        grid=(nb,),
        in_specs=[
            pl.BlockSpec((bm, 408), lambda i: (i, 0)),
            full(m1.shape), full(b1v.shape), full(m2.shape), full(b2v.shape),
            full(m3.shape), full(b3v.shape),
        ],
        out_specs=pl.BlockSpec((bm, 96), lambda i: (i, 0)),
        compiler_params=pltpu.CompilerParams(
            vmem_limit_bytes=100 * 1024 * 1024),
    )(xf, m1, b1v, m2, b2v, m3, b3v)


def _dinv_bodyfn(a_ref, b_ref, di_ref, di2_ref):
    d = a_ref[:, 0:1] + b_ref[:, 0:1]
    inv = jnp.where(d > 0, lax.rsqrt(jnp.maximum(d, 1e-12)), 0.0)
    di_ref[...] = inv
    di2_ref[...] = inv * inv


def _dinv(dega, degb):
    return pl.pallas_call(
        _dinv_bodyfn,
        out_shape=[jax.ShapeDtypeStruct((N, 1), jnp.float32),
                   jax.ShapeDtypeStruct((N, 1), jnp.float32)],
        grid=(N // _BM,),
        in_specs=[pl.BlockSpec((_BM, 16), lambda i: (i, 0)),
                  pl.BlockSpec((_BM, 16), lambda i: (i, 0))],
        out_specs=[pl.BlockSpec((_BM, 1), lambda i: (i, 0)),
                   pl.BlockSpec((_BM, 1), lambda i: (i, 0))],
    )(dega, degb)


def _prep_bodyfn(C, F, x_ref, d_ref, t_ref):
    xs = x_ref[...] * d_ref[...]
    for c in range(C):
        t_ref[c] = xs[:, c * F:(c + 1) * F]


def _prep0(xp, dinv, C, F):
    return pl.pallas_call(
        functools.partial(_prep_bodyfn, C, F),
        out_shape=jax.ShapeDtypeStruct((C, S, F), jnp.float32),
        grid=(N // _BM,),
        in_specs=[pl.BlockSpec((_BM, C * F), lambda i: (i, 0)),
                  pl.BlockSpec((_BM, 1), lambda i: (i, 0))],
        out_specs=pl.BlockSpec((C, _BM, F), lambda i: (0, i, 0)),
    )(xp, dinv)


def _scale2_bodyfn(z_ref, d_ref, d2_ref, h_ref, t_ref):
    h_ref[0] = z_ref[0] * d_ref[...]
    t_ref[0] = z_ref[0] * d2_ref[...]


def _scale1_bodyfn(z_ref, d_ref, h_ref):
    h_ref[0] = z_ref[0] * d_ref[...]


def _scale(z, dinv, dinv2, C, F):
    spec3 = pl.BlockSpec((1, _BM, F), lambda c, i: (c, i, 0))
    spec1 = pl.BlockSpec((_BM, 1), lambda c, i: (i, 0))
    if dinv2 is None:
        return pl.pallas_call(
            _scale1_bodyfn,
            out_shape=jax.ShapeDtypeStruct((C, S, F), jnp.float32),
            grid=(C, N // _BM),
            in_specs=[spec3, spec1],
            out_specs=spec3,
        )(z, dinv)
    return pl.pallas_call(
        _scale2_bodyfn,
        out_shape=[jax.ShapeDtypeStruct((C, S, F), jnp.float32),
                   jax.ShapeDtypeStruct((C, S, F), jnp.float32)],
        grid=(C, N // _BM),
        in_specs=[spec3, spec1, spec1],
        out_specs=[spec3, spec3],
    )(z, dinv, dinv2)


def _mm_bodyfn(x_ref, w_ref, o_ref):
    o_ref[...] = jnp.dot(x_ref[...], w_ref[...],
                         preferred_element_type=jnp.float32)


def _dense_mm(x, w):
    k, dout = w.shape
    return pl.pallas_call(
        _mm_bodyfn,
        out_shape=jax.ShapeDtypeStruct((N, dout), jnp.float32),
        grid=(N // _BM,),
        in_specs=[pl.BlockSpec((_BM, k), lambda i: (i, 0)),
                  pl.BlockSpec((k, dout), lambda i: (0, 0))],
        out_specs=pl.BlockSpec((_BM, dout), lambda i: (i, 0)),
    )(x, w)


def _cmm_bodyfn(C, h_ref, w_ref, o_ref):
    acc = jnp.dot(h_ref[0], w_ref[0], preferred_element_type=jnp.float32)
    for c in range(1, C):
        acc = acc + jnp.dot(h_ref[c], w_ref[c],
                            preferred_element_type=jnp.float32)
    o_ref[...] = acc


def _chunk_mm(h, wc):
    C, F, dout = wc.shape
    return pl.pallas_call(
        functools.partial(_cmm_bodyfn, C),
        out_shape=jax.ShapeDtypeStruct((N, dout), jnp.float32),
        grid=(N // _BM,),
        in_specs=[pl.BlockSpec((C, _BM, F), lambda i: (0, i, 0)),
                  pl.BlockSpec((C, F, dout), lambda i: (0, 0, 0))],
        out_specs=pl.BlockSpec((_BM, dout), lambda i: (i, 0)),
    )(h, wc)


def _comb_tanh_bodyfn(a_ref, b_ref, c_ref, bias_ref, o_ref):
    o_ref[...] = jnp.tanh(a_ref[...] + b_ref[...] + c_ref[...] + bias_ref[...])


def _comb_smax_bodyfn(a_ref, b_ref, c_ref, bias_ref, o_ref):
    x = a_ref[...] + b_ref[...] + c_ref[...] + bias_ref[...]
    m = jnp.max(x, axis=-1, keepdims=True)
    e = jnp.exp(x - m)
    o_ref[...] = e / jnp.sum(e, axis=-1, keepdims=True)


def _combine(p0, p1, p2, bias, softmax):
    dout = p0.shape[1]
    body = _comb_smax_bodyfn if softmax else _comb_tanh_bodyfn
    return pl.pallas_call(
        body,
        out_shape=jax.ShapeDtypeStruct((N, dout), jnp.float32),
        grid=(N // _BM,),
        in_specs=[pl.BlockSpec((_BM, dout), lambda i: (i, 0))] * 3
                 + [pl.BlockSpec((1, dout), lambda i: (0, 0))],
        out_specs=pl.BlockSpec((_BM, dout), lambda i: (i, 0)),
    )(p0, p1, p2, bias)


def _band(hi, i, a):
    return (jnp.arange(hi)[:, None, None] - jnp.arange(i)[None, :, None]
            == jnp.arange(a)[None, None, :]).astype(jnp.float32)


def _conv_mats(w1, b1, w2, b2, w3, b3):
    m1 = jnp.einsum('hia,wjb,oab->hwoij', _band(8, 6, 3), _band(51, 49, 3),
                    w1[:, 0]).reshape(408, 2940)
    m2 = jnp.einsum('hia,wjb,ocab->chwoij', _band(6, 4, 3), _band(49, 48, 2),
                    w2).reshape(2940, 3840)
    m3 = jnp.einsum('hia,wjb,ocab->chwoij', _band(4, 2, 3), _band(48, 47, 2),
                    w3).reshape(3840, 94)
    m3 = jnp.pad(m3, ((0, 0), (0, 2)))
    b1v = jnp.repeat(b1, 294)[None]
    b2v = jnp.repeat(b2, 192)[None]
    b3v = jnp.pad(jnp.repeat(b3, 94), (0, 2))[None]
    return m1, b1v, m2, b2v, m3, b3v


def _tag_layer(x, dinv, dinv2, w, b, eip, zeros, softmax):
    din, dout = w.shape[1], w.shape[2]
    F = 24
    C = (din + F - 1) // F
    wp = jnp.pad(w, ((0, 0), (0, C * F - din), (0, 0)))
    xp = jnp.pad(x, ((0, 0), (0, C * F - din)))
    hop = _make_hop(C, F)
    p0 = _dense_mm(xp, wp[0])
    table0 = _prep0(xp, dinv, C, F)
    z1 = hop(table0.reshape(C * S, F), eip, zeros)
    h1, table1 = _scale(z1.reshape(C, S, F), dinv, dinv2, C, F)
    p1 = _chunk_mm(h1, wp[1].reshape(C, F, dout))
    z2 = hop(table1.reshape(C * S, F), eip, zeros)
    h2 = _scale(z2.reshape(C, S, F), dinv, None, C, F)
    p2 = _chunk_mm(h2, wp[2].reshape(C, F, dout))
    return _combine(p0, p1, p2, b[None], softmax)


def kernel(inputs, edge_index, w1, b1, w2, b2, w3, b3,
           tag1_w, tag1_b, tag2_w, tag2_b, tag3_w, tag3_b):
    pad = jnp.concatenate([jnp.zeros((1, EPAD - E), jnp.int32),
                           jnp.full((1, EPAD - E), N, jnp.int32)])
    eip = jnp.concatenate([edge_index, pad], axis=1)
    zeros16 = jnp.zeros((ZPT, 16), jnp.float32)
    zeros24 = jnp.zeros((ZPT, 24), jnp.float32)
    ones16 = jnp.ones((128, 16), jnp.float32)

    degz = _make_deg()(eip, zeros16, ones16)
    dinv, dinv2 = _dinv(degz[:N], degz[S:S + N])

    m1, b1v, m2, b2v, m3, b3v = _conv_mats(w1, b1, w2, b2, w3, b3)
    x = _encoder(inputs.reshape(N, 408), m1, b1v, m2, b2v, m3, b3v)

    t1w = jnp.pad(tag1_w, ((0, 0), (0, 2), (0, 0)))
    x = _tag_layer(x, dinv, dinv2, t1w, tag1_b, eip, zeros24, False)
    x = _tag_layer(x, dinv, dinv2, tag2_w, tag2_b, eip, zeros24, False)
    return _tag_layer(x, dinv, dinv2, tag3_w, tag3_b, eip, zeros24, True)


# speedup vs baseline: 4.3959x; 1.1931x over previous
import functools

import jax
import jax.numpy as jnp
from jax import lax
from jax.experimental import pallas as pl
from jax.experimental.pallas import tpu as pltpu
from jax.experimental.pallas import tpu_sc as plsc

N = 50000
E = 800000
EPAD = 819200
ACC_ROWS = 50048
S = ACC_ROWS
ZPT = ACC_ROWS // 16
P = 8


def _hop_body(C, table, eip, zeros, z, sbuf, dbuf, gbuf, rows, acc,
              sem_i, sem_g, sem_s):
    t = lax.axis_index("s")
    core = lax.axis_index("c")
    ept = EPAD // 16
    iters = ept // (P * 128)
    for cc in range((C + 1) // 2):
        c = 2 * cc + core

        @pl.when(c < C)
        def _():
            cn = c * S
            pltpu.sync_copy(zeros, acc.at[pl.ds(t * ZPT, ZPT)])
            plsc.subcore_barrier()

            def body(j, carry):
                ebase = t * ept + j * (P * 128)
                d_i = []
                for p in range(P):
                    d_i.append(pltpu.async_copy(
                        eip.at[0, pl.ds(ebase + p * 128, 128)], sbuf.at[p],
                        sem_i))
                    d_i.append(pltpu.async_copy(
                        eip.at[1, pl.ds(ebase + p * 128, 128)], dbuf.at[p],
                        sem_i))
                d_g = []
                for p in range(P):
                    d_i[2 * p].wait()
                    d_i[2 * p + 1].wait()
                    for k in range(8):
                        gbuf[p, pl.ds(k * 16, 16)] = (
                            sbuf[p, pl.ds(k * 16, 16)] + cn)
                    d_g.append(pltpu.async_copy(
                        table.at[gbuf.at[p]], rows.at[p], sem_g))
                d_s = []
                for p in range(P):
                    d_g[p].wait()
                    d_s.append(pltpu.async_copy(
                        rows.at[p], acc.at[dbuf.at[p]], sem_s, add=True))
                for d in d_s:
                    d.wait()
                return carry

            lax.fori_loop(0, iters, body, 0)
            plsc.subcore_barrier()
            pltpu.sync_copy(acc.at[pl.ds(t * ZPT, ZPT)],
                            z.at[pl.ds(cn + t * ZPT, ZPT)])
            plsc.subcore_barrier()


@functools.lru_cache(maxsize=None)
def _make_hop(C, F):
    mesh = plsc.VectorSubcoreMesh(core_axis_name="c", subcore_axis_name="s")
    return pl.kernel(
        functools.partial(_hop_body, C),
        out_type=jax.ShapeDtypeStruct((C * S, F), jnp.float32),
        mesh=mesh,
        compiler_params=pltpu.CompilerParams(use_tc_tiling_on_sc=False),
        scratch_types=[
            pltpu.VMEM((P, 128), jnp.int32),
            pltpu.VMEM((P, 128), jnp.int32),
            pltpu.VMEM((P, 128), jnp.int32),
            pltpu.VMEM((P, 128, F), jnp.float32),
            pltpu.VMEM_SHARED((ACC_ROWS, F), jnp.float32),
            pltpu.SemaphoreType.DMA,
            pltpu.SemaphoreType.DMA,
            pltpu.SemaphoreType.DMA,
        ],
    )


def _deg_body(eip, zeros, ones, degz, dbuf, ones_v, acc, sem_i, sem_s):
    t = lax.axis_index("s")
    core = lax.axis_index("c")
    ept = EPAD // 32
    iters = ept // (P * 128)
    pltpu.sync_copy(zeros, acc.at[pl.ds(t * ZPT, ZPT)])
    pltpu.sync_copy(ones, ones_v)
    plsc.subcore_barrier()

    def body(j, carry):
        ebase = core * (EPAD // 2) + t * ept + j * (P * 128)
        d_i = [pltpu.async_copy(eip.at[1, pl.ds(ebase + p * 128, 128)],
                                dbuf.at[p], sem_i) for p in range(P)]
        d_s = []
        for p in range(P):
            d_i[p].wait()
            d_s.append(pltpu.async_copy(ones_v, acc.at[dbuf.at[p]], sem_s,
                                        add=True))
        for d in d_s:
            d.wait()
        return carry

    lax.fori_loop(0, iters, body, 0)
    plsc.subcore_barrier()
    pltpu.sync_copy(acc.at[pl.ds(t * ZPT, ZPT)],
                    degz.at[pl.ds(core * S + t * ZPT, ZPT)])


@functools.lru_cache(maxsize=None)
def _make_deg():
    mesh = plsc.VectorSubcoreMesh(core_axis_name="c", subcore_axis_name="s")
    return pl.kernel(
        _deg_body,
        out_type=jax.ShapeDtypeStruct((2 * S, 16), jnp.float32),
        mesh=mesh,
        compiler_params=pltpu.CompilerParams(use_tc_tiling_on_sc=False),
        scratch_types=[
            pltpu.VMEM((P, 128), jnp.int32),
            pltpu.VMEM((128, 16), jnp.float32),
            pltpu.VMEM_SHARED((ACC_ROWS, 16), jnp.float32),
            pltpu.SemaphoreType.DMA,
            pltpu.SemaphoreType.DMA,
        ],
    )


_BM = 1000


def _enc_bodyfn(x_ref, m1_ref, b1_ref, m2_ref, b2_ref, m3_ref, b3_ref, o_ref):
    h = jnp.dot(x_ref[...], m1_ref[...], preferred_element_type=jnp.float32)
    h = h + b1_ref[...]
    h = jnp.where(h > 0, h, 0.01 * h)
    h = jnp.dot(h, m2_ref[...], preferred_element_type=jnp.float32)
    h = h + b2_ref[...]
    h = jnp.where(h > 0, h, 0.01 * h)
    h = jnp.dot(h, m3_ref[...], preferred_element_type=jnp.float32)
    o_ref[...] = jnp.tanh(h + b3_ref[...])


def _encoder(xf, m1, b1v, m2, b2v, m3, b3v):
    bm = 400
    nb = N // bm
    full = lambda shape: pl.BlockSpec(shape, lambda i: tuple(0 for _ in shape))
    return pl.pallas_call(
        _enc_bodyfn,
        out_shape=jax.ShapeDtypeStruct((N, 96), jnp.float32),
        grid=(nb,),
        in_specs=[
            pl.BlockSpec((bm, 408), lambda i: (i, 0)),
            full(m1.shape), full(b1v.shape), full(m2.shape), full(b2v.shape),
            full(m3.shape), full(b3v.shape),
        ],
        out_specs=pl.BlockSpec((bm, 96), lambda i: (i, 0)),
        compiler_params=pltpu.CompilerParams(
            vmem_limit_bytes=100 * 1024 * 1024),
    )(xf, m1, b1v, m2, b2v, m3, b3v)


def _dinv_bodyfn(a_ref, b_ref, di_ref, di2_ref):
    d = a_ref[:, 0:1] + b_ref[:, 0:1]
    inv = jnp.where(d > 0, lax.rsqrt(jnp.maximum(d, 1e-12)), 0.0)
    di_ref[...] = inv
    di2_ref[...] = inv * inv


def _dinv(dega, degb):
    return pl.pallas_call(
        _dinv_bodyfn,
        out_shape=[jax.ShapeDtypeStruct((N, 1), jnp.float32),
                   jax.ShapeDtypeStruct((N, 1), jnp.float32)],
        grid=(N // _BM,),
        in_specs=[pl.BlockSpec((_BM, 16), lambda i: (i, 0)),
                  pl.BlockSpec((_BM, 16), lambda i: (i, 0))],
        out_specs=[pl.BlockSpec((_BM, 1), lambda i: (i, 0)),
                   pl.BlockSpec((_BM, 1), lambda i: (i, 0))],
    )(dega, degb)


def _prep_bodyfn(C, F, x_ref, d_ref, t_ref):
    xs = x_ref[...] * d_ref[...]
    for c in range(C):
        t_ref[c] = xs[:, c * F:(c + 1) * F]


def _prep0(xp, dinv, C, F):
    return pl.pallas_call(
        functools.partial(_prep_bodyfn, C, F),
        out_shape=jax.ShapeDtypeStruct((C, S, F), jnp.float32),
        grid=(N // _BM,),
        in_specs=[pl.BlockSpec((_BM, C * F), lambda i: (i, 0)),
                  pl.BlockSpec((_BM, 1), lambda i: (i, 0))],
        out_specs=pl.BlockSpec((C, _BM, F), lambda i: (0, i, 0)),
    )(xp, dinv)


def _scale2_bodyfn(z_ref, d_ref, d2_ref, h_ref, t_ref):
    h_ref[0] = z_ref[0] * d_ref[...]
    t_ref[0] = z_ref[0] * d2_ref[...]


def _scale1_bodyfn(z_ref, d_ref, h_ref):
    h_ref[0] = z_ref[0] * d_ref[...]


def _scale(z, dinv, dinv2, C, F):
    spec3 = pl.BlockSpec((1, _BM, F), lambda c, i: (c, i, 0))
    spec1 = pl.BlockSpec((_BM, 1), lambda c, i: (i, 0))
    if dinv2 is None:
        return pl.pallas_call(
            _scale1_bodyfn,
            out_shape=jax.ShapeDtypeStruct((C, S, F), jnp.float32),
            grid=(C, N // _BM),
            in_specs=[spec3, spec1],
            out_specs=spec3,
        )(z, dinv)
    return pl.pallas_call(
        _scale2_bodyfn,
        out_shape=[jax.ShapeDtypeStruct((C, S, F), jnp.float32),
                   jax.ShapeDtypeStruct((C, S, F), jnp.float32)],
        grid=(C, N // _BM),
        in_specs=[spec3, spec1, spec1],
        out_specs=[spec3, spec3],
    )(z, dinv, dinv2)


def _mm_bodyfn(x_ref, w_ref, o_ref):
    o_ref[...] = jnp.dot(x_ref[...], w_ref[...],
                         preferred_element_type=jnp.float32)


def _dense_mm(x, w):
    k, dout = w.shape
    return pl.pallas_call(
        _mm_bodyfn,
        out_shape=jax.ShapeDtypeStruct((N, dout), jnp.float32),
        grid=(N // _BM,),
        in_specs=[pl.BlockSpec((_BM, k), lambda i: (i, 0)),
                  pl.BlockSpec((k, dout), lambda i: (0, 0))],
        out_specs=pl.BlockSpec((_BM, dout), lambda i: (i, 0)),
    )(x, w)


def _cmm_bodyfn(C, h_ref, w_ref, o_ref):
    acc = jnp.dot(h_ref[0], w_ref[0], preferred_element_type=jnp.float32)
    for c in range(1, C):
        acc = acc + jnp.dot(h_ref[c], w_ref[c],
                            preferred_element_type=jnp.float32)
    o_ref[...] = acc


def _chunk_mm(h, wc):
    C, F, dout = wc.shape
    return pl.pallas_call(
        functools.partial(_cmm_bodyfn, C),
        out_shape=jax.ShapeDtypeStruct((N, dout), jnp.float32),
        grid=(N // _BM,),
        in_specs=[pl.BlockSpec((C, _BM, F), lambda i: (0, i, 0)),
                  pl.BlockSpec((C, F, dout), lambda i: (0, 0, 0))],
        out_specs=pl.BlockSpec((_BM, dout), lambda i: (i, 0)),
    )(h, wc)


def _comb_tanh_bodyfn(a_ref, b_ref, c_ref, bias_ref, o_ref):
    o_ref[...] = jnp.tanh(a_ref[...] + b_ref[...] + c_ref[...] + bias_ref[...])


def _comb_smax_bodyfn(a_ref, b_ref, c_ref, bias_ref, o_ref):
    x = a_ref[...] + b_ref[...] + c_ref[...] + bias_ref[...]
    m = jnp.max(x, axis=-1, keepdims=True)
    e = jnp.exp(x - m)
    o_ref[...] = e / jnp.sum(e, axis=-1, keepdims=True)


def _combine(p0, p1, p2, bias, softmax):
    dout = p0.shape[1]
    body = _comb_smax_bodyfn if softmax else _comb_tanh_bodyfn
    return pl.pallas_call(
        body,
        out_shape=jax.ShapeDtypeStruct((N, dout), jnp.float32),
        grid=(N // _BM,),
        in_specs=[pl.BlockSpec((_BM, dout), lambda i: (i, 0))] * 3
                 + [pl.BlockSpec((1, dout), lambda i: (0, 0))],
        out_specs=pl.BlockSpec((_BM, dout), lambda i: (i, 0)),
    )(p0, p1, p2, bias)


def _band(hi, i, a):
    return (jnp.arange(hi)[:, None, None] - jnp.arange(i)[None, :, None]
            == jnp.arange(a)[None, None, :]).astype(jnp.float32)


def _conv_mats(w1, b1, w2, b2, w3, b3):
    m1 = jnp.einsum('hia,wjb,oab->hwoij', _band(8, 6, 3), _band(51, 49, 3),
                    w1[:, 0]).reshape(408, 2940)
    m2 = jnp.einsum('hia,wjb,ocab->chwoij', _band(6, 4, 3), _band(49, 48, 2),
                    w2).reshape(2940, 3840)
    m3 = jnp.einsum('hia,wjb,ocab->chwoij', _band(4, 2, 3), _band(48, 47, 2),
                    w3).reshape(3840, 94)
    m3 = jnp.pad(m3, ((0, 0), (0, 2)))
    b1v = jnp.repeat(b1, 294)[None]
    b2v = jnp.repeat(b2, 192)[None]
    b3v = jnp.pad(jnp.repeat(b3, 94), (0, 2))[None]
    return m1, b1v, m2, b2v, m3, b3v


def _tag_layer1(x, dinv, dinv2, w, b, eip, zeros):
    din, dout = w.shape[1], w.shape[2]
    F = 24
    C = din // F
    hop = _make_hop(C, F)
    p0 = _dense_mm(x, w[0])
    table0 = _prep0(x, dinv, C, F)
    z1 = hop(table0.reshape(C * S, F), eip, zeros)
    h1, table1 = _scale(z1.reshape(C, S, F), dinv, dinv2, C, F)
    p1 = _chunk_mm(h1, w[1].reshape(C, F, dout))
    z2 = hop(table1.reshape(C * S, F), eip, zeros)
    h2 = _scale(z2.reshape(C, S, F), dinv, None, C, F)
    p2 = _chunk_mm(h2, w[2].reshape(C, F, dout))
    return _combine(p0, p1, p2, b[None], False)


def _tag_layer2(x, dinv, w, b, eip, zeros):
    dout = w.shape[2]
    F = 24
    C = 6
    wp = jnp.pad(w, ((0, 0), (0, C * F - w.shape[1]), (0, 0)))
    xp = jnp.pad(x, ((0, 0), (0, C * F - x.shape[1])))
    p0 = _dense_mm(xp, wp[0])
    table0 = _prep0(xp, dinv, C, F)
    z1 = _make_hop(C, F)(table0.reshape(C * S, F), eip, zeros)
    h1 = _scale(z1.reshape(C, S, F), dinv, None, C, F)
    p1 = _chunk_mm(h1, wp[1].reshape(C, F, dout))
    y = _chunk_mm(h1, wp[2].reshape(C, F, dout))
    C2 = 3
    yp = jnp.pad(y, ((0, 0), (0, C2 * F - dout)))
    ty = _prep0(yp, dinv, C2, F)
    z2 = _make_hop(C2, F)(ty.reshape(C2 * S, F), eip, zeros)
    h2 = _scale(z2.reshape(C2, S, F), dinv, None, C2, F)
    p2 = _chunk_mm(h2, jnp.eye(C2 * F, dout).reshape(C2, F, dout))
    return _combine(p0, p1, p2, b[None], False)


def _tag_layer3(x, dinv, dinv2, w, b, eip, zeros):
    dout = w.shape[2]
    F = 16
    p0 = _dense_mm(x, w[0])
    y12 = _dense_mm(x, jnp.concatenate([w[1], w[2]], axis=1))
    t0 = _prep0(y12, dinv, 2, F)
    z = _make_hop(2, F)(t0.reshape(2 * S, F), eip, zeros)
    h, tab = _scale(z.reshape(2, S, F), dinv, dinv2, 2, F)
    sel0 = jnp.concatenate([jnp.eye(F), jnp.zeros((F, F))]).reshape(2, F, F)
    p1 = _chunk_mm(h, sel0)
    z2 = _make_hop(1, F)(tab[1], eip, zeros)
    h2 = _scale(z2.reshape(1, S, F), dinv, None, 1, F)
    p2 = _chunk_mm(h2, jnp.eye(F).reshape(1, F, F))
    return _combine(p0, p1, p2, b[None], True)


def kernel(inputs, edge_index, w1, b1, w2, b2, w3, b3,
           tag1_w, tag1_b, tag2_w, tag2_b, tag3_w, tag3_b):
    pad = jnp.concatenate([jnp.zeros((1, EPAD - E), jnp.int32),
                           jnp.full((1, EPAD - E), N, jnp.int32)])
    eip = jnp.concatenate([edge_index, pad], axis=1)
    zeros16 = jnp.zeros((ZPT, 16), jnp.float32)
    zeros24 = jnp.zeros((ZPT, 24), jnp.float32)
    ones16 = jnp.ones((128, 16), jnp.float32)

    degz = _make_deg()(eip, zeros16, ones16)
    dinv, dinv2 = _dinv(degz[:N], degz[S:S + N])

    m1, b1v, m2, b2v, m3, b3v = _conv_mats(w1, b1, w2, b2, w3, b3)
    x = _encoder(inputs.reshape(N, 408), m1, b1v, m2, b2v, m3, b3v)

    t1w = jnp.pad(tag1_w, ((0, 0), (0, 2), (0, 0)))
    x = _tag_layer1(x, dinv, dinv2, t1w, tag1_b, eip, zeros24)
    x = _tag_layer2(x, dinv, tag2_w, tag2_b, eip, zeros24)
    return _tag_layer3(x, dinv, dinv2, tag3_w, tag3_b, eip, zeros16)
